# Initial kernel scaffold; baseline (speedup 1.0000x reference)
import jax
import jax.numpy as jnp
from jax.experimental import pallas as pl


def kernel(edge_index, edge_rel, ent_emb, rel_emb, a_self, a_neigh, a_rel):
    raise NotImplementedError("write your pallas kernel here")


# speedup vs baseline: 7.2534x; 7.2534x over previous
import jax
import jax.numpy as jnp
from jax import lax
from jax.experimental import pallas as pl
from jax.experimental.pallas import tpu as pltpu
from jax.experimental.pallas import tpu_sc as plsc

N = 10000
NH = 100
R = 1000
E = 320000
D = 200
CW = 64
NR = 10016
RPT = 664
RPT_LAST = NR - 15 * RPT
EP = 323584
CH = 128
CHUNKS = EP // 16 // CH

_MESH = plsc.VectorSubcoreMesh(core_axis_name="c", subcore_axis_name="s")
_SC_PARAMS = pltpu.CompilerParams(use_tc_tiling_on_sc=False,
                                  needs_layout_passes=False)
F32 = jnp.float32
I32 = jnp.int32


def _tile_row_copy(src, dst, s, src_base, dst_base):
    off = s * RPT

    @pl.when(s < 15)
    def _():
        pltpu.sync_copy(src.at[pl.ds(src_base + off, RPT)],
                        dst.at[pl.ds(dst_base + off, RPT)])

    @pl.when(s == 15)
    def _():
        pltpu.sync_copy(src.at[pl.ds(src_base + 15 * RPT, RPT_LAST)],
                        dst.at[pl.ds(dst_base + 15 * RPT, RPT_LAST)])


def _k1_body(gidx, dstp, tcomb, zeros, out, acc, gi2, di2, rows, sem):
    c = lax.axis_index("c")
    s = lax.axis_index("s")
    _tile_row_copy(zeros, acc, s, 0, 0)
    plsc.subcore_barrier()
    base0 = s * (CHUNKS * CH)

    def chunk(k, carry):
        b = base0 + k * CH
        pltpu.sync_copy(gidx.at[pl.ds(c * EP + b, CH)], gi2.at[0])
        pltpu.sync_copy(dstp.at[pl.ds(b, CH)], di2.at[0])
        pltpu.async_copy(tcomb.at[gi2.at[0]], rows, sem).wait()
        pltpu.sync_copy(rows, acc.at[di2.at[0]], add=True)
        return carry

    lax.fori_loop(0, CHUNKS, chunk, 0)
    plsc.subcore_barrier()
    _tile_row_copy(acc, out, s, 0, c * NR)


def _attention_e(r0t, r1t, s0t, s1t, n0t, n1t, srcv, dstv, relv):
    r0v = plsc.load_gather(r0t, [relv])
    r1v = plsc.load_gather(r1t, [relv])
    s0v = plsc.load_gather(s0t, [dstv])
    s1v = plsc.load_gather(s1t, [dstv])
    n0v = plsc.load_gather(n0t, [srcv])
    n1v = plsc.load_gather(n1t, [srcv])
    a0 = r0v + s0v + n0v
    a1 = r1v + s1v + n1v
    a0 = jnp.where(a0 > 0.0, a0, a0 * 0.01)
    a1 = jnp.where(a1 > 0.0, a1, a1 * 0.01)
    return jnp.exp(a0), jnp.exp(a1)


def _k2_body(srcp, dstp, relp, scal, ra, zeros, out,
             acc, s0t, s1t, n0t, n1t, r0t, r1t, stage, si2, di2, ri2, sem):
    c = lax.axis_index("c")
    s = lax.axis_index("s")
    _tile_row_copy(zeros, acc, s, 0, 0)
    pltpu.sync_copy(scal.at[0], s0t)
    pltpu.sync_copy(scal.at[1], s1t)
    pltpu.sync_copy(scal.at[2], n0t)
    pltpu.sync_copy(scal.at[3], n1t)
    pltpu.sync_copy(ra.at[0], r0t)
    pltpu.sync_copy(ra.at[1], r1t)

    zv = jnp.zeros((16,), F32)

    def zrow(j, carry):
        stage[j, :] = zv
        return carry

    lax.fori_loop(0, CH, zrow, 0)
    plsc.subcore_barrier()

    wid = c * 16 + s
    base0 = wid * (EP // 32)
    col0 = jnp.zeros((16,), I32)
    col1 = jnp.ones((16,), I32)
    lane = lax.iota(I32, 16)

    def chunk(k, carry):
        b = base0 + k * CH
        pltpu.sync_copy(srcp.at[pl.ds(b, CH)], si2.at[0])
        pltpu.sync_copy(dstp.at[pl.ds(b, CH)], di2.at[0])
        pltpu.sync_copy(relp.at[pl.ds(b, CH)], ri2.at[0])
        for sub in range(8):
            sl = pl.ds(sub * 16, 16)
            e0, e1 = _attention_e(r0t, r1t, s0t, s1t, n0t, n1t,
                                  si2[0, sl], di2[0, sl], ri2[0, sl])
            ridx = sub * 16 + lane
            plsc.store_scatter(stage, [ridx, col0], e0)
            plsc.store_scatter(stage, [ridx, col1], e1)
        pltpu.sync_copy(stage, acc.at[di2.at[0]], add=True)
        return carry

    lax.fori_loop(0, EP // 32 // CH, chunk, 0)
    plsc.subcore_barrier()
    _tile_row_copy(acc, out, s, 0, c * NR)


def _k3_body(srcp, dstp, relp, scal, ra, iat, fcat, zeros, out,
             acc, s0t, s1t, n0t, n1t, r0t, r1t, ia0t, ia1t,
             rows, wbuf, si2, gi2, di2, ri2, sem):
    c = lax.axis_index("c")
    s = lax.axis_index("s")
    _tile_row_copy(zeros, acc, s, 0, 0)
    pltpu.sync_copy(scal.at[0], s0t)
    pltpu.sync_copy(scal.at[1], s1t)
    pltpu.sync_copy(scal.at[2], n0t)
    pltpu.sync_copy(scal.at[3], n1t)
    pltpu.sync_copy(ra.at[0], r0t)
    pltpu.sync_copy(ra.at[1], r1t)
    pltpu.sync_copy(iat.at[0], ia0t)
    pltpu.sync_copy(iat.at[1], ia1t)
    plsc.subcore_barrier()

    base0 = s * (CHUNKS * CH)
    cn16 = jnp.broadcast_to(c * N, (16,)).astype(I32)

    def chunk(k, carry):
        b = base0 + k * CH
        pltpu.sync_copy(srcp.at[pl.ds(b, CH)], si2.at[0])
        pltpu.sync_copy(dstp.at[pl.ds(b, CH)], di2.at[0])
        pltpu.sync_copy(relp.at[pl.ds(b, CH)], ri2.at[0])
        for sub in range(8):
            sl = pl.ds(sub * 16, 16)
            gi2[0, sl] = si2[0, sl] + cn16
        cp = pltpu.async_copy(fcat.at[gi2.at[0]], rows, sem)
        for sub in range(8):
            sl = pl.ds(sub * 16, 16)
            dstv = di2[0, sl]
            e0, e1 = _attention_e(r0t, r1t, s0t, s1t, n0t, n1t,
                                  si2[0, sl], dstv, ri2[0, sl])
            i0 = plsc.load_gather(ia0t, [dstv])
            i1 = plsc.load_gather(ia1t, [dstv])
            wbuf[sl] = e0 * i0 + e1 * i1
        cp.wait()

        def rowloop(j, carry2):
            wj = plsc.load_gather(wbuf, [jnp.broadcast_to(j, (16,)).astype(I32)])
            for cc in range(CW // 16):
                slc = pl.ds(cc * 16, 16)
                rows[j, slc] = rows[j, slc] * wj
            return carry2

        lax.fori_loop(0, CH, rowloop, 0)
        pltpu.sync_copy(rows, acc.at[di2.at[0]], add=True)
        return carry

    lax.fori_loop(0, CHUNKS, chunk, 0)
    plsc.subcore_barrier()
    _tile_row_copy(acc, out, s, 0, c * NR)


_k1 = pl.kernel(
    _k1_body,
    out_type=jax.ShapeDtypeStruct((2 * NR, CW), F32),
    mesh=_MESH,
    compiler_params=_SC_PARAMS,
    scratch_types=[
        pltpu.VMEM_SHARED((NR, CW), F32),
        pltpu.VMEM((1, CH), I32),
        pltpu.VMEM((1, CH), I32),
        pltpu.VMEM((CH, CW), F32),
        pltpu.SemaphoreType.DMA,
    ],
)

_k2 = pl.kernel(
    _k2_body,
    out_type=jax.ShapeDtypeStruct((2 * NR, 16), F32),
    mesh=_MESH,
    compiler_params=_SC_PARAMS,
    scratch_types=[
        pltpu.VMEM_SHARED((NR, 16), F32),
        pltpu.VMEM((NR,), F32),
        pltpu.VMEM((NR,), F32),
        pltpu.VMEM((NR,), F32),
        pltpu.VMEM((NR,), F32),
        pltpu.VMEM((R,), F32),
        pltpu.VMEM((R,), F32),
        pltpu.VMEM((CH, 16), F32),
        pltpu.VMEM((1, CH), I32),
        pltpu.VMEM((1, CH), I32),
        pltpu.VMEM((1, CH), I32),
        pltpu.SemaphoreType.DMA,
    ],
)

_k3 = pl.kernel(
    _k3_body,
    out_type=jax.ShapeDtypeStruct((2 * NR, CW), F32),
    mesh=_MESH,
    compiler_params=_SC_PARAMS,
    scratch_types=[
        pltpu.VMEM_SHARED((NR, CW), F32),
        pltpu.VMEM((NR,), F32),
        pltpu.VMEM((NR,), F32),
        pltpu.VMEM((NR,), F32),
        pltpu.VMEM((NR,), F32),
        pltpu.VMEM((R,), F32),
        pltpu.VMEM((R,), F32),
        pltpu.VMEM((NR,), F32),
        pltpu.VMEM((NR,), F32),
        pltpu.VMEM((CH, CW), F32),
        pltpu.VMEM((CH,), F32),
        pltpu.VMEM((1, CH), I32),
        pltpu.VMEM((1, CH), I32),
        pltpu.VMEM((1, CH), I32),
        pltpu.VMEM((1, CH), I32),
        pltpu.SemaphoreType.DMA,
    ],
)


def _ep1_kernel(acca_ref, accb_ref, o_ref):
    a = acca_ref[...].reshape(2, NR, CW)
    b = accb_ref[...].reshape(2, NR, CW)
    ent = jnp.concatenate([a[0, :N, :], b[0, :N, :NH - CW]], axis=1)
    rel = jnp.concatenate([a[1, :N, :], b[1, :N, :NH - CW]], axis=1)
    deg = jnp.maximum(b[0, :N, NH - CW], 1.0)[:, None]
    feat = jnp.concatenate([ent, rel], axis=1) / deg
    o_ref[...] = jnp.maximum(feat, 0.0)


def _ep3_kernel(agga_ref, aggb_ref, o_ref):
    a = agga_ref[...].reshape(2, NR, CW)
    b = aggb_ref[...].reshape(2, NR, CW)
    feat = jnp.concatenate(
        [a[0, :N, :], a[1, :N, :], b[0, :N, :], b[1, :N, :D - 3 * CW]], axis=1)
    o_ref[...] = jnp.maximum(feat, 0.0)


_ep1 = pl.pallas_call(_ep1_kernel, out_shape=jax.ShapeDtypeStruct((N, D), F32))
_ep3 = pl.pallas_call(_ep3_kernel, out_shape=jax.ShapeDtypeStruct((N, D), F32))


def kernel(edge_index, edge_rel, ent_emb, rel_emb, a_self, a_neigh, a_rel):
    src = edge_index[0].astype(I32)
    dst = edge_index[1].astype(I32)
    rel = edge_rel.astype(I32)
    pad = EP - E
    src_p = jnp.concatenate([src, jnp.zeros((pad,), I32)])
    dst_p = jnp.concatenate([dst, jnp.full((pad,), N, I32)])
    rel_p = jnp.concatenate([rel, jnp.zeros((pad,), I32)])
    gidx = jnp.concatenate([src_p, rel_p + N])
    zer = jnp.zeros((NR, CW), F32)

    emb = jnp.concatenate([ent_emb, rel_emb], axis=0)
    tca = jnp.asarray(emb[:, :CW])
    tcb = jnp.zeros((N + R, CW), F32).at[:, :NH - CW].set(emb[:, CW:])
    tcb = tcb.at[:N, NH - CW].set(1.0)

    acca = _k1(gidx, dst_p, tca, zer)
    accb = _k1(gidx, dst_p, tcb, zer)
    feat = _ep1(acca, accb)
    outs = [feat]
    for _ in range(2):
        s_sc = feat @ a_self.T
        n_sc = feat @ a_neigh.T
        r_sc = rel_emb @ a_rel.T
        scal = jnp.zeros((4, NR), F32)
        scal = scal.at[0, :N].set(s_sc[:, 0]).at[1, :N].set(s_sc[:, 1])
        scal = scal.at[2, :N].set(n_sc[:, 0]).at[3, :N].set(n_sc[:, 1])
        ra = jnp.asarray(r_sc.T)

        asum2 = _k2(src_p, dst_p, rel_p, scal, ra, jnp.zeros((NR, 16), F32))
        asum = asum2.reshape(2, NR, 16).sum(0)[:N, :2]
        ia = 0.5 / (asum + 1e-16)
        iat = jnp.zeros((2, NR), F32).at[:, :N].set(ia.T)

        fq = jnp.zeros((N, 4 * CW), F32).at[:, :D].set(feat)
        fcat_a = jnp.concatenate([fq[:, 0 * CW:1 * CW], fq[:, 1 * CW:2 * CW]])
        fcat_b = jnp.concatenate([fq[:, 2 * CW:3 * CW], fq[:, 3 * CW:4 * CW]])

        agga = _k3(src_p, dst_p, rel_p, scal, ra, iat, fcat_a, zer)
        aggb = _k3(src_p, dst_p, rel_p, scal, ra, iat, fcat_b, zer)
        feat = _ep3(agga, aggb)
        outs.append(feat)
    return jnp.concatenate(outs, axis=1)


# speedup vs baseline: 10.6540x; 1.4688x over previous
import jax
import jax.numpy as jnp
from jax import lax
from jax.experimental import pallas as pl
from jax.experimental.pallas import tpu as pltpu
from jax.experimental.pallas import tpu_sc as plsc

N = 10000
NH = 100
R = 1000
E = 320000
D = 200
CW = 64
NR = 10016
RPT = 664
RPT_LAST = NR - 15 * RPT
SUP = 2048
EP = 327680
NSUP = EP // SUP
CH = 128
NCH = SUP // CH
SUP_T1 = NSUP // 16
SUP_T2 = NSUP // 32

_MESH = plsc.VectorSubcoreMesh(core_axis_name="c", subcore_axis_name="s")
_SC_PARAMS = pltpu.CompilerParams(use_tc_tiling_on_sc=False,
                                  needs_layout_passes=False)
F32 = jnp.float32
I32 = jnp.int32


def _tile_row_copy(src, dst, s, src_base, dst_base):
    off = s * RPT

    @pl.when(s < 15)
    def _():
        pltpu.sync_copy(src.at[pl.ds(src_base + off, RPT)],
                        dst.at[pl.ds(dst_base + off, RPT)])

    @pl.when(s == 15)
    def _():
        pltpu.sync_copy(src.at[pl.ds(src_base + 15 * RPT, RPT_LAST)],
                        dst.at[pl.ds(dst_base + 15 * RPT, RPT_LAST)])


def _vcopy128(dst2, eb, src_off, add16=None):
    for sub in range(8):
        v = eb[pl.ds(src_off + sub * 16, 16)]
        if add16 is not None:
            v = v + add16
        dst2[0, pl.ds(sub * 16, 16)] = v


def _k1_body(e3, tcomb, zeros, out, acc, eb, gi2a, gi2b, di2a, di2b,
             rows0, rows1, gsem, ssem):
    c = lax.axis_index("c")
    s = lax.axis_index("s")
    _tile_row_copy(zeros, acc, s, 0, 0)
    plsc.subcore_barrier()
    cblk = c * (2 * SUP)
    cn16 = jnp.broadcast_to(c * N, (16,)).astype(I32)
    gi2 = (gi2a, gi2b)
    di2 = (di2a, di2b)
    rows = (rows0, rows1)

    def super_chunk(g, carry):
        pltpu.sync_copy(e3.at[pl.ds((s * SUP_T1 + g) * (3 * SUP), 3 * SUP)], eb)
        scat = [None, None]

        def prep(j):
            p = j & 1
            _vcopy128(gi2[p], eb, cblk + j * CH, cn16)
            _vcopy128(di2[p], eb, SUP + j * CH)

        def gather(j):
            p = j & 1
            return pltpu.async_copy(tcomb.at[gi2[p].at[0]], rows[p], gsem)

        prep(0)
        gd = gather(0)
        for j in range(NCH):
            p = j & 1
            gd_next = None
            if j + 1 < NCH:
                if scat[1 - p] is not None:
                    scat[1 - p].wait()
                    scat[1 - p] = None
                prep(j + 1)
                gd_next = gather(j + 1)
            gd.wait()
            scat[p] = pltpu.async_copy(rows[p], acc.at[di2[p].at[0]], ssem,
                                       add=True)
            gd = gd_next
        for p in range(2):
            if scat[p] is not None:
                scat[p].wait()
        return carry

    lax.fori_loop(0, SUP_T1, super_chunk, 0)
    plsc.subcore_barrier()
    _tile_row_copy(acc, out, s, 0, c * NR)


def _attention_e(r0t, r1t, s0t, s1t, n0t, n1t, srcv, dstv, relv):
    r0v = plsc.load_gather(r0t, [relv])
    r1v = plsc.load_gather(r1t, [relv])
    s0v = plsc.load_gather(s0t, [dstv])
    s1v = plsc.load_gather(s1t, [dstv])
    n0v = plsc.load_gather(n0t, [srcv])
    n1v = plsc.load_gather(n1t, [srcv])
    a0 = r0v + s0v + n0v
    a1 = r1v + s1v + n1v
    a0 = jnp.where(a0 > 0.0, a0, a0 * 0.01)
    a1 = jnp.where(a1 > 0.0, a1, a1 * 0.01)
    return jnp.exp(a0), jnp.exp(a1)


def _k2_body(e3, scal, ra, zeros, out,
             acc, s0t, s1t, n0t, n1t, r0t, r1t, eb,
             stage0, stage1, di2a, di2b, ssem):
    c = lax.axis_index("c")
    s = lax.axis_index("s")
    _tile_row_copy(zeros, acc, s, 0, 0)
    pltpu.sync_copy(scal.at[0], s0t)
    pltpu.sync_copy(scal.at[1], s1t)
    pltpu.sync_copy(scal.at[2], n0t)
    pltpu.sync_copy(scal.at[3], n1t)
    pltpu.sync_copy(ra.at[0], r0t)
    pltpu.sync_copy(ra.at[1], r1t)

    zv = jnp.zeros((16,), F32)
    stage = (stage0, stage1)
    di2 = (di2a, di2b)

    def zrow(j, carry):
        stage0[j, :] = zv
        stage1[j, :] = zv
        return carry

    lax.fori_loop(0, CH, zrow, 0)
    plsc.subcore_barrier()

    wid = c * 16 + s
    col0 = jnp.zeros((16,), I32)
    col1 = jnp.ones((16,), I32)
    lane = lax.iota(I32, 16)

    def super_chunk(g, carry):
        pltpu.sync_copy(e3.at[pl.ds((wid * SUP_T2 + g) * (3 * SUP), 3 * SUP)],
                        eb)
        scat = [None, None]
        for j in range(NCH):
            p = j & 1
            if scat[p] is not None:
                scat[p].wait()
                scat[p] = None
            _vcopy128(di2[p], eb, SUP + j * CH)
            for sub in range(8):
                o = j * CH + sub * 16
                e0, e1 = _attention_e(
                    r0t, r1t, s0t, s1t, n0t, n1t,
                    eb[pl.ds(o, 16)], eb[pl.ds(SUP + o, 16)],
                    eb[pl.ds(2 * SUP + o, 16)])
                ridx = sub * 16 + lane
                plsc.store_scatter(stage[p], [ridx, col0], e0)
                plsc.store_scatter(stage[p], [ridx, col1], e1)
            scat[p] = pltpu.async_copy(stage[p], acc.at[di2[p].at[0]], ssem,
                                       add=True)
        for p in range(2):
            if scat[p] is not None:
                scat[p].wait()
        return carry

    lax.fori_loop(0, SUP_T2, super_chunk, 0)
    plsc.subcore_barrier()
    _tile_row_copy(acc, out, s, 0, c * NR)


def _k3_body(e3, scal, ra, iat, fcat, zeros, out,
             acc, s0t, s1t, n0t, n1t, r0t, r1t, ia0t, ia1t,
             eb, wbuf, gi2a, gi2b, di2a, di2b, rows0, rows1, gsem, ssem):
    c = lax.axis_index("c")
    s = lax.axis_index("s")
    _tile_row_copy(zeros, acc, s, 0, 0)
    pltpu.sync_copy(scal.at[0], s0t)
    pltpu.sync_copy(scal.at[1], s1t)
    pltpu.sync_copy(scal.at[2], n0t)
    pltpu.sync_copy(scal.at[3], n1t)
    pltpu.sync_copy(ra.at[0], r0t)
    pltpu.sync_copy(ra.at[1], r1t)
    pltpu.sync_copy(iat.at[0], ia0t)
    pltpu.sync_copy(iat.at[1], ia1t)
    plsc.subcore_barrier()

    cn16 = jnp.broadcast_to(c * N, (16,)).astype(I32)
    gi2 = (gi2a, gi2b)
    di2 = (di2a, di2b)
    rows = (rows0, rows1)

    def super_chunk(g, carry):
        pltpu.sync_copy(e3.at[pl.ds((s * SUP_T1 + g) * (3 * SUP), 3 * SUP)], eb)
        scat = [None, None]

        def prep(j):
            p = j & 1
            _vcopy128(gi2[p], eb, j * CH, cn16)
            _vcopy128(di2[p], eb, SUP + j * CH)

        def gather(j):
            p = j & 1
            return pltpu.async_copy(fcat.at[gi2[p].at[0]], rows[p], gsem)

        prep(0)
        gd = gather(0)
        for j in range(NCH):
            p = j & 1
            gd_next = None
            if j + 1 < NCH:
                if scat[1 - p] is not None:
                    scat[1 - p].wait()
                    scat[1 - p] = None
                prep(j + 1)
                gd_next = gather(j + 1)
            for sub in range(8):
                o = j * CH + sub * 16
                dstv = eb[pl.ds(SUP + o, 16)]
                e0, e1 = _attention_e(
                    r0t, r1t, s0t, s1t, n0t, n1t,
                    eb[pl.ds(o, 16)], dstv, eb[pl.ds(2 * SUP + o, 16)])
                i0 = plsc.load_gather(ia0t, [dstv])
                i1 = plsc.load_gather(ia1t, [dstv])
                wbuf[pl.ds(sub * 16, 16)] = e0 * i0 + e1 * i1
            gd.wait()
            rp = rows[p]

            def rowloop(jj, carry2):
                wj = plsc.load_gather(
                    wbuf, [jnp.broadcast_to(jj, (16,)).astype(I32)])
                for cc in range(CW // 16):
                    slc = pl.ds(cc * 16, 16)
                    rp[jj, slc] = rp[jj, slc] * wj
                return carry2

            lax.fori_loop(0, CH, rowloop, 0)
            scat[p] = pltpu.async_copy(rows[p], acc.at[di2[p].at[0]], ssem,
                                       add=True)
            gd = gd_next
        for p in range(2):
            if scat[p] is not None:
                scat[p].wait()
        return carry

    lax.fori_loop(0, SUP_T1, super_chunk, 0)
    plsc.subcore_barrier()
    _tile_row_copy(acc, out, s, 0, c * NR)


_IDX2 = pltpu.VMEM((1, CH), I32)

_k1 = pl.kernel(
    _k1_body,
    out_type=jax.ShapeDtypeStruct((2 * NR, CW), F32),
    mesh=_MESH,
    compiler_params=_SC_PARAMS,
    scratch_types=[
        pltpu.VMEM_SHARED((NR, CW), F32),
        pltpu.VMEM((3 * SUP,), I32),
        _IDX2, _IDX2, _IDX2, _IDX2,
        pltpu.VMEM((CH, CW), F32),
        pltpu.VMEM((CH, CW), F32),
        pltpu.SemaphoreType.DMA,
        pltpu.SemaphoreType.DMA,
    ],
)

_k2 = pl.kernel(
    _k2_body,
    out_type=jax.ShapeDtypeStruct((2 * NR, 16), F32),
    mesh=_MESH,
    compiler_params=_SC_PARAMS,
    scratch_types=[
        pltpu.VMEM_SHARED((NR, 16), F32),
        pltpu.VMEM((NR,), F32),
        pltpu.VMEM((NR,), F32),
        pltpu.VMEM((NR,), F32),
        pltpu.VMEM((NR,), F32),
        pltpu.VMEM((R,), F32),
        pltpu.VMEM((R,), F32),
        pltpu.VMEM((3 * SUP,), I32),
        pltpu.VMEM((CH, 16), F32),
        pltpu.VMEM((CH, 16), F32),
        _IDX2, _IDX2,
        pltpu.SemaphoreType.DMA,
    ],
)

_k3 = pl.kernel(
    _k3_body,
    out_type=jax.ShapeDtypeStruct((2 * NR, CW), F32),
    mesh=_MESH,
    compiler_params=_SC_PARAMS,
    scratch_types=[
        pltpu.VMEM_SHARED((NR, CW), F32),
        pltpu.VMEM((NR,), F32),
        pltpu.VMEM((NR,), F32),
        pltpu.VMEM((NR,), F32),
        pltpu.VMEM((NR,), F32),
        pltpu.VMEM((R,), F32),
        pltpu.VMEM((R,), F32),
        pltpu.VMEM((NR,), F32),
        pltpu.VMEM((NR,), F32),
        pltpu.VMEM((3 * SUP,), I32),
        pltpu.VMEM((CH,), F32),
        _IDX2, _IDX2, _IDX2, _IDX2,
        pltpu.VMEM((CH, CW), F32),
        pltpu.VMEM((CH, CW), F32),
        pltpu.SemaphoreType.DMA,
        pltpu.SemaphoreType.DMA,
    ],
)


def _ep1_kernel(acca_ref, accb_ref, o_ref):
    a = acca_ref[...].reshape(2, NR, CW)
    b = accb_ref[...].reshape(2, NR, CW)
    ent = jnp.concatenate([a[0, :N, :], b[0, :N, :NH - CW]], axis=1)
    rel = jnp.concatenate([a[1, :N, :], b[1, :N, :NH - CW]], axis=1)
    deg = jnp.maximum(b[0, :N, NH - CW], 1.0)[:, None]
    feat = jnp.concatenate([ent, rel], axis=1) / deg
    o_ref[...] = jnp.maximum(feat, 0.0)


def _ep3_kernel(agga_ref, aggb_ref, o_ref):
    a = agga_ref[...].reshape(2, NR, CW)
    b = aggb_ref[...].reshape(2, NR, CW)
    feat = jnp.concatenate(
        [a[0, :N, :], a[1, :N, :], b[0, :N, :], b[1, :N, :D - 3 * CW]], axis=1)
    o_ref[...] = jnp.maximum(feat, 0.0)


_ep1 = pl.pallas_call(_ep1_kernel, out_shape=jax.ShapeDtypeStruct((N, D), F32))
_ep3 = pl.pallas_call(_ep3_kernel, out_shape=jax.ShapeDtypeStruct((N, D), F32))


def kernel(edge_index, edge_rel, ent_emb, rel_emb, a_self, a_neigh, a_rel):
    src = edge_index[0].astype(I32)
    dst = edge_index[1].astype(I32)
    rel = edge_rel.astype(I32)
    pad = EP - E
    src_p = jnp.concatenate([src, jnp.zeros((pad,), I32)])
    dst_p = jnp.concatenate([dst, jnp.full((pad,), N, I32)])
    rel_p = jnp.concatenate([rel, jnp.zeros((pad,), I32)])
    e3 = jnp.stack([src_p.reshape(NSUP, SUP), dst_p.reshape(NSUP, SUP),
                    rel_p.reshape(NSUP, SUP)], axis=1).reshape(-1)
    zer = jnp.zeros((NR, CW), F32)

    emb = jnp.concatenate([ent_emb, rel_emb], axis=0)
    tca = jnp.asarray(emb[:, :CW])
    tcb = jnp.zeros((N + R, CW), F32).at[:, :NH - CW].set(emb[:, CW:])
    tcb = tcb.at[:N, NH - CW].set(1.0)

    acca = _k1(e3, tca, zer)
    accb = _k1(e3, tcb, zer)
    feat = _ep1(acca, accb)
    outs = [feat]
    for _ in range(2):
        s_sc = feat @ a_self.T
        n_sc = feat @ a_neigh.T
        r_sc = rel_emb @ a_rel.T
        scal = jnp.zeros((4, NR), F32)
        scal = scal.at[0, :N].set(s_sc[:, 0]).at[1, :N].set(s_sc[:, 1])
        scal = scal.at[2, :N].set(n_sc[:, 0]).at[3, :N].set(n_sc[:, 1])
        ra = jnp.asarray(r_sc.T)

        asum2 = _k2(e3, scal, ra, jnp.zeros((NR, 16), F32))
        asum = asum2.reshape(2, NR, 16).sum(0)[:N, :2]
        ia = 0.5 / (asum + 1e-16)
        iat = jnp.zeros((2, NR), F32).at[:, :N].set(ia.T)

        fq = jnp.zeros((N, 4 * CW), F32).at[:, :D].set(feat)
        fcat_a = jnp.concatenate([fq[:, 0 * CW:1 * CW], fq[:, 1 * CW:2 * CW]])
        fcat_b = jnp.concatenate([fq[:, 2 * CW:3 * CW], fq[:, 3 * CW:4 * CW]])

        agga = _k3(e3, scal, ra, iat, fcat_a, zer)
        aggb = _k3(e3, scal, ra, iat, fcat_b, zer)
        feat = _ep3(agga, aggb)
        outs.append(feat)
    return jnp.concatenate(outs, axis=1)


# speedup vs baseline: 12.5219x; 1.1753x over previous
import jax
import jax.numpy as jnp
from jax import lax
from jax.experimental import pallas as pl
from jax.experimental.pallas import tpu as pltpu
from jax.experimental.pallas import tpu_sc as plsc

N = 10000
NH = 100
R = 1000
E = 320000
D = 200
CW1 = 128
CW3 = 64
NR = 10016
NR1 = 10008
RPT1 = 664
RPT1_LAST = NR1 - 15 * RPT1
NR2 = 1256
RPT2 = 80
RPT2_LAST = NR2 - 15 * RPT2
NR3 = 10000
RPT3 = 664
RPT3_LAST = NR3 - 15 * RPT3
SUP = 2048
EP = 327680
NSUP = EP // SUP
CH = 128
NCH = SUP // CH
SUP_T1 = NSUP // 16
SUP_T2 = NSUP // 32

_MESH = plsc.VectorSubcoreMesh(core_axis_name="c", subcore_axis_name="s")
_SC_PARAMS = pltpu.CompilerParams(use_tc_tiling_on_sc=False,
                                  needs_layout_passes=False)
F32 = jnp.float32
I32 = jnp.int32


def _zero_buf(buf, width):
    zv = jnp.zeros((16,), F32)

    def zrow(j, carry):
        for cc in range(width // 16):
            buf[j, pl.ds(cc * 16, 16)] = zv
        return carry

    lax.fori_loop(0, CH, zrow, 0)


def _self_zero(acc, zbuf, s, rpt, last):
    nfull, rem = rpt // CH, rpt % CH

    @pl.when(s < 15)
    def _():
        for k in range(nfull):
            pltpu.sync_copy(zbuf, acc.at[pl.ds(s * rpt + k * CH, CH)])
        if rem:
            pltpu.sync_copy(zbuf.at[pl.ds(0, rem)],
                            acc.at[pl.ds(s * rpt + nfull * CH, rem)])

    @pl.when(s == 15)
    def _():
        pltpu.sync_copy(zbuf.at[pl.ds(0, last)] if last <= CH else zbuf,
                        acc.at[pl.ds(15 * rpt, last)])


def _tile_row_copy(src, dst, s, src_base, dst_base, rpt, last):
    off = s * rpt

    @pl.when(s < 15)
    def _():
        pltpu.sync_copy(src.at[pl.ds(src_base + off, rpt)],
                        dst.at[pl.ds(dst_base + off, rpt)])

    @pl.when(s == 15)
    def _():
        pltpu.sync_copy(src.at[pl.ds(src_base + 15 * rpt, last)],
                        dst.at[pl.ds(dst_base + 15 * rpt, last)])


def _vcopy128(dst2, eb, src_off, add16=None, shift3=False, clampn=False):
    for sub in range(8):
        v = eb[pl.ds(src_off + sub * 16, 16)]
        if add16 is not None:
            v = v + add16
        if shift3:
            v = lax.shift_right_logical(v, 3)
        if clampn:
            v = jnp.where(v < N, v, 0)
        dst2[0, pl.ds(sub * 16, 16)] = v


def _k1_body(e3, tcomb, tok, out, acc, eb, gi2a, gi2b, di2a, di2b,
             rows0, rows1, gsem, ssem):
    c = lax.axis_index("c")
    s = lax.axis_index("s")
    _zero_buf(rows0, CW1)
    _self_zero(acc, rows0, s, RPT1, RPT1_LAST)
    plsc.subcore_barrier()
    cblk = c * (2 * SUP)
    cn16 = jnp.broadcast_to(c * N, (16,)).astype(I32)
    gi2 = (gi2a, gi2b)
    di2 = (di2a, di2b)
    rows = (rows0, rows1)

    def super_chunk(g, carry):
        pltpu.sync_copy(e3.at[pl.ds((s * SUP_T1 + g) * (3 * SUP), 3 * SUP)], eb)
        scat = [None, None]

        def prep(j):
            p = j & 1
            _vcopy128(gi2[p], eb, cblk + j * CH, cn16)
            _vcopy128(di2[p], eb, SUP + j * CH)

        def gather(j):
            p = j & 1
            return pltpu.async_copy(tcomb.at[gi2[p].at[0]], rows[p], gsem)

        prep(0)
        gd = gather(0)
        for j in range(NCH):
            p = j & 1
            gd_next = None
            if j + 1 < NCH:
                if scat[1 - p] is not None:
                    scat[1 - p].wait()
                    scat[1 - p] = None
                prep(j + 1)
                gd_next = gather(j + 1)
            gd.wait()
            scat[p] = pltpu.async_copy(rows[p], acc.at[di2[p].at[0]], ssem,
                                       add=True)
            gd = gd_next
        for p in range(2):
            if scat[p] is not None:
                scat[p].wait()
        return carry

    lax.fori_loop(0, SUP_T1, super_chunk, 0)
    plsc.subcore_barrier()
    _tile_row_copy(acc, out, s, 0, c * NR1, RPT1, RPT1_LAST)


def _attention_e(r0t, r1t, s0t, s1t, n0t, n1t, srcv, dstv, relv):
    r0v = plsc.load_gather(r0t, [relv])
    r1v = plsc.load_gather(r1t, [relv])
    s0v = plsc.load_gather(s0t, [dstv])
    s1v = plsc.load_gather(s1t, [dstv])
    n0v = plsc.load_gather(n0t, [srcv])
    n1v = plsc.load_gather(n1t, [srcv])
    a0 = r0v + s0v + n0v
    a1 = r1v + s1v + n1v
    a0 = jnp.where(a0 > 0.0, a0, a0 * 0.01)
    a1 = jnp.where(a1 > 0.0, a1, a1 * 0.01)
    return jnp.exp(a0), jnp.exp(a1)


def _k2_body(e3, scal, ra, tok, out,
             acc, s0t, s1t, n0t, n1t, r0t, r1t, eb,
             stage0, stage1, di2a, di2b, ssem):
    c = lax.axis_index("c")
    s = lax.axis_index("s")
    _zero_buf(stage0, 16)
    _zero_buf(stage1, 16)
    _self_zero(acc, stage0, s, RPT2, RPT2_LAST)
    pltpu.sync_copy(scal.at[0], s0t)
    pltpu.sync_copy(scal.at[1], s1t)
    pltpu.sync_copy(scal.at[2], n0t)
    pltpu.sync_copy(scal.at[3], n1t)
    pltpu.sync_copy(ra.at[0], r0t)
    pltpu.sync_copy(ra.at[1], r1t)
    plsc.subcore_barrier()

    stage = (stage0, stage1)
    di2 = (di2a, di2b)
    wid = c * 16 + s
    lane = lax.iota(I32, 16)

    def super_chunk(g, carry):
        pltpu.sync_copy(e3.at[pl.ds((wid * SUP_T2 + g) * (3 * SUP), 3 * SUP)],
                        eb)
        scat = [None, None]
        for j in range(NCH):
            p = j & 1
            if scat[p] is not None:
                scat[p].wait()
                scat[p] = None
            _vcopy128(di2[p], eb, SUP + j * CH, shift3=True)
            for sub in range(8):
                o = j * CH + sub * 16
                dstv = eb[pl.ds(SUP + o, 16)]
                e0, e1 = _attention_e(
                    r0t, r1t, s0t, s1t, n0t, n1t,
                    eb[pl.ds(o, 16)], dstv,
                    eb[pl.ds(2 * SUP + o, 16)])
                ridx = sub * 16 + lane
                colv = (dstv & 7) * 2
                plsc.store_scatter(stage[p], [ridx, colv], e0)
                plsc.store_scatter(stage[p], [ridx, colv + 1], e1)
            scat[p] = pltpu.async_copy(stage[p], acc.at[di2[p].at[0]], ssem,
                                       add=True)
        for p in range(2):
            if scat[p] is not None:
                scat[p].wait()
        return carry

    lax.fori_loop(0, SUP_T2, super_chunk, 0)
    plsc.subcore_barrier()
    _tile_row_copy(acc, out, s, 0, c * NR2, RPT2, RPT2_LAST)


def _k3_body(e3, scal, ra, iat, fcat, tok, out,
             acc, s0t, s1t, n0t, n1t, r0t, r1t, ia0t, ia1t,
             eb, wbuf, gi2a, gi2b, di2a, di2b, rows0, rows1, gsem, ssem):
    c = lax.axis_index("c")
    s = lax.axis_index("s")
    _zero_buf(rows0, CW3)
    _self_zero(acc, rows0, s, RPT3, RPT3_LAST)
    pltpu.sync_copy(scal.at[0], s0t)
    pltpu.sync_copy(scal.at[1], s1t)
    pltpu.sync_copy(scal.at[2], n0t)
    pltpu.sync_copy(scal.at[3], n1t)
    pltpu.sync_copy(ra.at[0], r0t)
    pltpu.sync_copy(ra.at[1], r1t)
    pltpu.sync_copy(iat.at[0], ia0t)
    pltpu.sync_copy(iat.at[1], ia1t)
    plsc.subcore_barrier()

    cn16 = jnp.broadcast_to(c * N, (16,)).astype(I32)
    gi2 = (gi2a, gi2b)
    di2 = (di2a, di2b)
    rows = (rows0, rows1)

    def super_chunk(g, carry):
        pltpu.sync_copy(e3.at[pl.ds((s * SUP_T1 + g) * (3 * SUP), 3 * SUP)], eb)
        scat = [None, None]

        def prep(j):
            p = j & 1
            _vcopy128(gi2[p], eb, j * CH, cn16)
            _vcopy128(di2[p], eb, SUP + j * CH, clampn=True)

        def gather(j):
            p = j & 1
            return pltpu.async_copy(fcat.at[gi2[p].at[0]], rows[p], gsem)

        prep(0)
        gd = gather(0)
        for j in range(NCH):
            p = j & 1
            gd_next = None
            if j + 1 < NCH:
                if scat[1 - p] is not None:
                    scat[1 - p].wait()
                    scat[1 - p] = None
                prep(j + 1)
                gd_next = gather(j + 1)
            for sub in range(8):
                o = j * CH + sub * 16
                dstv = eb[pl.ds(SUP + o, 16)]
                e0, e1 = _attention_e(
                    r0t, r1t, s0t, s1t, n0t, n1t,
                    eb[pl.ds(o, 16)], dstv, eb[pl.ds(2 * SUP + o, 16)])
                i0 = plsc.load_gather(ia0t, [dstv])
                i1 = plsc.load_gather(ia1t, [dstv])
                wbuf[pl.ds(sub * 16, 16)] = e0 * i0 + e1 * i1
            gd.wait()
            rp = rows[p]

            def rowloop(j2, carry2):
                jj = j2 * 2
                w0 = plsc.load_gather(
                    wbuf, [jnp.broadcast_to(jj, (16,)).astype(I32)])
                w1 = plsc.load_gather(
                    wbuf, [jnp.broadcast_to(jj + 1, (16,)).astype(I32)])
                for cc in range(CW3 // 16):
                    slc = pl.ds(cc * 16, 16)
                    rp[jj, slc] = rp[jj, slc] * w0
                for cc in range(CW3 // 16):
                    slc = pl.ds(cc * 16, 16)
                    rp[jj + 1, slc] = rp[jj + 1, slc] * w1
                return carry2

            lax.fori_loop(0, CH // 2, rowloop, 0)
            scat[p] = pltpu.async_copy(rows[p], acc.at[di2[p].at[0]], ssem,
                                       add=True)
            gd = gd_next
        for p in range(2):
            if scat[p] is not None:
                scat[p].wait()
        return carry

    lax.fori_loop(0, SUP_T1, super_chunk, 0)
    plsc.subcore_barrier()
    _tile_row_copy(acc, out, s, 0, c * NR3, RPT3, RPT3_LAST)


_IDX2 = pltpu.VMEM((1, CH), I32)

_k1 = pl.kernel(
    _k1_body,
    out_type=jax.ShapeDtypeStruct((2 * NR1, CW1), F32),
    mesh=_MESH,
    compiler_params=_SC_PARAMS,
    scratch_types=[
        pltpu.VMEM_SHARED((NR1, CW1), F32),
        pltpu.VMEM((3 * SUP,), I32),
        _IDX2, _IDX2, _IDX2, _IDX2,
        pltpu.VMEM((CH, CW1), F32),
        pltpu.VMEM((CH, CW1), F32),
        pltpu.SemaphoreType.DMA,
        pltpu.SemaphoreType.DMA,
    ],
)

_k2 = pl.kernel(
    _k2_body,
    out_type=jax.ShapeDtypeStruct((2 * NR2, 16), F32),
    mesh=_MESH,
    compiler_params=_SC_PARAMS,
    scratch_types=[
        pltpu.VMEM_SHARED((NR2, 16), F32),
        pltpu.VMEM((NR,), F32),
        pltpu.VMEM((NR,), F32),
        pltpu.VMEM((NR,), F32),
        pltpu.VMEM((NR,), F32),
        pltpu.VMEM((R,), F32),
        pltpu.VMEM((R,), F32),
        pltpu.VMEM((3 * SUP,), I32),
        pltpu.VMEM((CH, 16), F32),
        pltpu.VMEM((CH, 16), F32),
        _IDX2, _IDX2,
        pltpu.SemaphoreType.DMA,
    ],
)

_k3 = pl.kernel(
    _k3_body,
    out_type=jax.ShapeDtypeStruct((2 * NR3, CW3), F32),
    mesh=_MESH,
    compiler_params=_SC_PARAMS,
    scratch_types=[
        pltpu.VMEM_SHARED((NR3, CW3), F32),
        pltpu.VMEM((NR,), F32),
        pltpu.VMEM((NR,), F32),
        pltpu.VMEM((NR,), F32),
        pltpu.VMEM((NR,), F32),
        pltpu.VMEM((R,), F32),
        pltpu.VMEM((R,), F32),
        pltpu.VMEM((NR,), F32),
        pltpu.VMEM((NR,), F32),
        pltpu.VMEM((3 * SUP,), I32),
        pltpu.VMEM((CH,), F32),
        _IDX2, _IDX2, _IDX2, _IDX2,
        pltpu.VMEM((CH, CW3), F32),
        pltpu.VMEM((CH, CW3), F32),
        pltpu.SemaphoreType.DMA,
        pltpu.SemaphoreType.DMA,
    ],
)


def _ep1_kernel(acc_ref, o_ref):
    a = acc_ref[...].reshape(2, NR1, CW1)
    deg = jnp.maximum(a[0, :N, NH], 1.0)[:, None]
    feat = jnp.concatenate([a[0, :N, :NH], a[1, :N, :NH]], axis=1) / deg
    o_ref[...] = jnp.maximum(feat, 0.0)


def _ep3_kernel(agga_ref, aggb_ref, o_ref):
    a = agga_ref[...].reshape(2, NR3, CW3)
    b = aggb_ref[...].reshape(2, NR3, CW3)
    feat = jnp.concatenate(
        [a[0], a[1], b[0], b[1, :, :D - 3 * CW3]], axis=1)
    o_ref[...] = jnp.maximum(feat, 0.0)


_ep1 = pl.pallas_call(_ep1_kernel, out_shape=jax.ShapeDtypeStruct((N, D), F32))
_ep3 = pl.pallas_call(_ep3_kernel, out_shape=jax.ShapeDtypeStruct((N, D), F32))


def _epcat_kernel(f0_ref, f1_ref, f2_ref, o_ref):
    o_ref[...] = jnp.concatenate([f0_ref[...], f1_ref[...], f2_ref[...]],
                                 axis=1)


_epcat = pl.pallas_call(_epcat_kernel,
                        out_shape=jax.ShapeDtypeStruct((N, 3 * D), F32))


def kernel(edge_index, edge_rel, ent_emb, rel_emb, a_self, a_neigh, a_rel):
    src = edge_index[0].astype(I32)
    dst = edge_index[1].astype(I32)
    rel = edge_rel.astype(I32)
    pad = EP - E
    src_p = jnp.concatenate([src, jnp.zeros((pad,), I32)])
    dst_p = jnp.concatenate([dst, jnp.full((pad,), N, I32)])
    rel_p = jnp.concatenate([rel, jnp.zeros((pad,), I32)])
    e3 = jnp.stack([src_p.reshape(NSUP, SUP), dst_p.reshape(NSUP, SUP),
                    rel_p.reshape(NSUP, SUP)], axis=1).reshape(-1)

    tcomb = jnp.zeros((N + R, CW1), F32)
    tcomb = tcomb.at[:N, :NH].set(ent_emb).at[:N, NH].set(1.0)
    tcomb = tcomb.at[N:, :NH].set(rel_emb)

    acc1 = _k1(e3, tcomb, jnp.zeros((128,), F32))
    feat = _ep1(acc1)
    tok = acc1.reshape(-1)[:128]
    outs = [feat]
    for _ in range(2):
        s_sc = feat @ a_self.T
        n_sc = feat @ a_neigh.T
        r_sc = rel_emb @ a_rel.T
        scal = jnp.zeros((4, NR), F32)
        scal = scal.at[0, :N].set(s_sc[:, 0]).at[1, :N].set(s_sc[:, 1])
        scal = scal.at[2, :N].set(n_sc[:, 0]).at[3, :N].set(n_sc[:, 1])
        ra = jnp.asarray(r_sc.T)

        asum2 = _k2(e3, scal, ra, tok)
        asum = asum2.reshape(2, NR2, 16).sum(0).reshape(NR2 * 8, 2)[:N]
        ia = 0.5 / (asum + 1e-16)
        iat = jnp.zeros((2, NR), F32).at[:, :N].set(ia.T)
        fq = jnp.zeros((N, 4 * CW3), F32).at[:, :D].set(feat)
        fcat_a = jnp.concatenate([fq[:, 0 * CW3:1 * CW3],
                                  fq[:, 1 * CW3:2 * CW3]])
        fcat_b = jnp.concatenate([fq[:, 2 * CW3:3 * CW3],
                                  fq[:, 3 * CW3:4 * CW3]])

        agga = _k3(e3, scal, ra, iat, fcat_a, asum2.reshape(-1)[:128])
        aggb = _k3(e3, scal, ra, iat, fcat_b, agga.reshape(-1)[:128])
        tok = aggb.reshape(-1)[:128]
        feat = _ep3(agga, aggb)
        outs.append(feat)
    return _epcat(*outs)
